# baseline (speedup 1.0000x reference)
import jax
import jax.numpy as jnp
from jax import lax
from jax.experimental import pallas as pl
from jax.experimental.pallas import tpu as pltpu
from jax.experimental.pallas import tpu_sc as plsc

B = 16384
SEQ = 50
PROG_VOCAB = 21
TERM_VOCAB = 1000
EMB = 32
NPOS = SEQ + 1

NW = 32
ROWS_PER_W = B // NW
ROWS_PER_BLK = 16
JPR = NPOS
IDX_PER_BLK = ROWS_PER_BLK * JPR
IDX_CHUNK = 48
NCHUNK = IDX_PER_BLK // IDX_CHUNK
BLK_PER_W = ROWS_PER_W // ROWS_PER_BLK


def _table_body(tt_ref, pp_ref, w_ref, b_ref, p_ref, out_ref):
    for g in range(1, NPOS):
        wp = jnp.dot(w_ref[pl.ds(EMB * g, EMB), :], p_ref[...],
                     preferred_element_type=jnp.float32)
        out_ref[pl.ds(TERM_VOCAB * g, TERM_VOCAB), :] = jnp.dot(
            tt_ref[...], wp,
            preferred_element_type=jnp.float32).astype(jnp.bfloat16)
    wp0 = jnp.dot(w_ref[pl.ds(0, EMB), :], p_ref[...],
                  preferred_element_type=jnp.float32)
    pe = jnp.dot(pp_ref[...], wp0,
                 preferred_element_type=jnp.float32) + jnp.dot(
                     b_ref[...], p_ref[...],
                     preferred_element_type=jnp.float32)
    out_ref[pl.ds(0, TERM_VOCAB), :] = jnp.zeros((TERM_VOCAB, EMB),
                                                 jnp.bfloat16)
    out_ref[pl.ds(0, 24), :] = pe.astype(jnp.bfloat16)
    out_ref[pl.ds(NPOS * TERM_VOCAB, 8), :] = jnp.zeros((8, EMB),
                                                        jnp.bfloat16)


NROWS = NPOS * TERM_VOCAB + 8
STAGE = NROWS // 16


def _build_table(terms_table, prog_pad, dense_w, b2, pmat):
    return pl.pallas_call(
        _table_body,
        out_shape=jax.ShapeDtypeStruct((NROWS, EMB), jnp.bfloat16),
    )(terms_table, prog_pad, dense_w, b2, pmat)


def _sc_body(table_hbm, terms_hbm, prog_hbm, out_hbm,
             table_sp, tt_v, pg_v, idx0, idx1, data0, data1, acc_v,
             sem0, sem1):
    sid = lax.axis_index("s")
    wid = sid * 2 + lax.axis_index("c")
    base_row = wid * ROWS_PER_W
    pltpu.sync_copy(table_hbm.at[pl.ds(sid * STAGE, STAGE)],
                    table_sp.at[pl.ds(sid * STAGE, STAGE)])
    pltpu.sync_copy(terms_hbm.at[pl.ds(base_row, ROWS_PER_W)], tt_v)
    pltpu.sync_copy(prog_hbm.at[pl.ds(base_row, ROWS_PER_W)], pg_v)
    lanes = lax.iota(jnp.int32, 16)
    plsc.subcore_barrier()

    def build_idx(bl, idx_v):
        r0 = bl * ROWS_PER_BLK
        nh = ROWS_PER_BLK // 16
        for h in range(nh):
            idx_v[pl.ds(h * 16, 16)] = pg_v[pl.ds(r0 + h * 16, 16)]
        for s in range(SEQ):
            off = jnp.int32((s + 1) * TERM_VOCAB)
            col = jnp.full((16,), s, jnp.int32)
            for h in range(nh):
                rows = r0 + h * 16 + lanes
                vals = plsc.load_gather(tt_v, [rows, col])
                idx_v[pl.ds((1 + s) * ROWS_PER_BLK + h * 16, 16)] = vals + off

    def fire(idx_v, data_v, sem):
        for c in range(NCHUNK):
            pltpu.async_copy(
                table_sp.at[idx_v.at[pl.ds(c * IDX_CHUNK, IDX_CHUNK)]],
                data_v.at[pl.ds(c * IDX_CHUNK, IDX_CHUNK)],
                sem)

    def drain(idx_v, data_v, sem):
        for c in range(NCHUNK):
            pltpu.make_async_copy(
                table_sp.at[idx_v.at[pl.ds(c * IDX_CHUNK, IDX_CHUNK)]],
                data_v.at[pl.ds(c * IDX_CHUNK, IDX_CHUNK)],
                sem).wait()

    def accum_out(bl, data_v):
        def grp_body(g, _):
            r0 = g * 8
            a0 = [jnp.zeros((16,), jnp.float32) for _ in range(8)]
            a1 = [jnp.zeros((16,), jnp.float32) for _ in range(8)]
            for j in range(JPR):
                for q in range(8):
                    vu = plsc.bitcast(
                        data_v[j * ROWS_PER_BLK + r0 + q, :], jnp.uint32)
                    a0[q] += plsc.bitcast(vu << jnp.uint32(16), jnp.float32)
                    a1[q] += plsc.bitcast(vu, jnp.float32)
            for q in range(8):
                acc_v[r0 + q, pl.ds(0, 16)] = jnp.maximum(a0[q], 0.0)
                acc_v[r0 + q, pl.ds(16, 16)] = jnp.maximum(a1[q], 0.0)
            return 0

        lax.fori_loop(0, ROWS_PER_BLK // 8, grp_body, 0)
        pltpu.sync_copy(
            acc_v,
            out_hbm.at[pl.ds(base_row + bl * ROWS_PER_BLK, ROWS_PER_BLK)])

    build_idx(0, idx0)
    fire(idx0, data0, sem0)

    def pair_body(h, _):
        a = 2 * h
        build_idx(a + 1, idx1)
        fire(idx1, data1, sem1)
        drain(idx0, data0, sem0)
        accum_out(a, data0)

        @pl.when(h < BLK_PER_W // 2 - 1)
        def _():
            build_idx(a + 2, idx0)
            fire(idx0, data0, sem0)

        drain(idx1, data1, sem1)
        accum_out(a + 1, data1)
        return 0

    lax.fori_loop(0, BLK_PER_W // 2, pair_body, 0)


def kernel(program_input, terms_input, program_table, terms_table, dense_w,
           dense_b):
    perm = jnp.stack(
        [jnp.arange(16, dtype=jnp.int32),
         16 + jnp.arange(16, dtype=jnp.int32)], axis=1).reshape(32)
    pmat = (jnp.arange(EMB, dtype=jnp.int32)[:, None]
            == perm[None, :]).astype(jnp.float32)

    prog_pad = jnp.pad(program_table, ((0, 24 - PROG_VOCAB), (0, 0)))
    b2 = dense_b.reshape(1, EMB)
    table = _build_table(terms_table, prog_pad, dense_w, b2, pmat)

    mesh = plsc.VectorSubcoreMesh(core_axis_name="c", subcore_axis_name="s")
    sc = pl.kernel(
        _sc_body,
        out_type=jax.ShapeDtypeStruct((B, EMB), jnp.float32),
        mesh=mesh,
        scratch_types=[
            pltpu.VMEM_SHARED((NROWS, EMB), jnp.bfloat16),
            pltpu.VMEM((ROWS_PER_W, SEQ), jnp.int32),
            pltpu.VMEM((ROWS_PER_W,), jnp.int32),
            pltpu.VMEM((IDX_PER_BLK,), jnp.int32),
            pltpu.VMEM((IDX_PER_BLK,), jnp.int32),
            pltpu.VMEM((IDX_PER_BLK, EMB), jnp.bfloat16),
            pltpu.VMEM((IDX_PER_BLK, EMB), jnp.bfloat16),
            pltpu.VMEM((ROWS_PER_BLK, EMB), jnp.float32),
            pltpu.SemaphoreType.DMA,
            pltpu.SemaphoreType.DMA,
        ],
        compiler_params=pltpu.CompilerParams(use_tc_tiling_on_sc=False,
                                             needs_layout_passes=False),
    )
    return sc(table, terms_input, program_input)
